# baseline (speedup 1.0000x reference)
import functools

import jax
import jax.numpy as jnp
from jax import lax
from jax.experimental import pallas as pl
from jax.experimental.pallas import tpu as pltpu
from jax.experimental.pallas import tpu_sc as plsc


def _router_body(x_ref, wr_ref, b_ref, topw_ref, tope_ref, r_ref, cnt_ref,
                 *, n_exp, tb):
    x = x_ref[...]
    wr = wr_ref[...]
    logits = lax.dot_general(
        x, wr, (((1,), (1,)), ((), ())), preferred_element_type=jnp.float32
    ) + b_ref[...]
    w = jax.nn.softmax(logits, axis=-1)
    lane = lax.broadcasted_iota(jnp.int32, w.shape, 1)
    rank = jnp.zeros_like(w)
    for j in range(n_exp):
        wj = w[:, j : j + 1]
        rank += (wj > w).astype(jnp.float32)
        rank += ((wj == w) & (j < lane)).astype(jnp.float32)
    on0 = rank < 0.5
    on1 = (rank >= 0.5) & (rank < 1.5)
    ew = jnp.exp(w)
    denom = jnp.sum(jnp.where(on0 | on1, ew, 0.0), axis=1, keepdims=True)
    w0 = jnp.sum(jnp.where(on0, ew, 0.0), axis=1, keepdims=True) / denom
    w1 = jnp.sum(jnp.where(on1, ew, 0.0), axis=1, keepdims=True) / denom
    e0 = jnp.sum(jnp.where(on0, lane, 0), axis=1, keepdims=True)
    e1 = jnp.sum(jnp.where(on1, lane, 0), axis=1, keepdims=True)
    topw_ref[...] = jnp.concatenate([w0, w1], axis=1)
    tope_ref[...] = jnp.concatenate([e0, e1], axis=1)
    occ = (on0 | on1).astype(jnp.bfloat16)
    row_i = lax.broadcasted_iota(jnp.int32, (tb, tb), 0)
    col_i = lax.broadcasted_iota(jnp.int32, (tb, tb), 1)
    ltr = (row_i > col_i).astype(jnp.bfloat16)
    cum = lax.dot_general(
        ltr, occ, (((1,), (0,)), ((), ())), preferred_element_type=jnp.float32
    )
    r0 = jnp.sum(jnp.where(on0, cum, 0.0), axis=1, keepdims=True)
    r1 = jnp.sum(jnp.where(on1, cum, 0.0), axis=1, keepdims=True)
    r_ref[...] = jnp.concatenate([r0, r1], axis=1).astype(jnp.int32)
    cnt_ref[...] = jnp.sum(occ.astype(jnp.float32), axis=0, keepdims=True)[None]


def _dispatch_body(cnt_ref, tope_ref, r_ref, p_ref, be_ref,
                   *, n_exp, tb, nt, nblk):
    t = pl.program_id(0)
    cnt = cnt_ref[:, 0, :]
    c = jnp.sum(cnt, axis=0, keepdims=True)
    row = lax.broadcasted_iota(jnp.int32, cnt.shape, 0)
    base = jnp.sum(jnp.where(row < t, cnt, 0.0), axis=0, keepdims=True)
    pb = jnp.floor((c + (tb - 1)) / tb)
    lane = lax.broadcasted_iota(jnp.int32, (1, n_exp), 1)
    start = jnp.zeros((1, n_exp), jnp.float32)
    for e in range(n_exp - 1):
        start += jnp.where(lane > e, pb[0:1, e : e + 1] * tb, 0.0)
    sb = start + base
    tope = tope_ref[...]
    r = r_ref[...].astype(jnp.float32)
    lane_tb = lax.broadcasted_iota(jnp.int32, (tope.shape[0], n_exp), 1)
    ps = []
    for k in range(2):
        onek = tope[:, k : k + 1] == lane_tb
        pk = jnp.sum(jnp.where(onek, sb, 0.0), axis=1, keepdims=True)
        ps.append(pk + r[:, k : k + 1])
    p_ref[...] = jnp.concatenate(ps, axis=1).astype(jnp.int32)

    @pl.when(t == 0)
    def _():
        lane_b = lax.broadcasted_iota(jnp.int32, (1, nblk), 1)
        acc = jnp.zeros((1, nblk), jnp.int32)
        run = pb[0:1, 0:1]
        for e in range(1, n_exp + 1):
            acc += (lane_b >= run.astype(jnp.int32)).astype(jnp.int32)
            if e < n_exp:
                run = run + pb[0:1, e : e + 1]
        be_ref[...] = acc[None]


def _sc_permute(p_flat, tok_flat, g_flat, dflt, padn):
    nk = p_flat.shape[0]
    mesh = plsc.VectorSubcoreMesh(core_axis_name="c", subcore_axis_name="s")

    @functools.partial(
        pl.kernel,
        out_type=[
            jax.ShapeDtypeStruct((padn,), jnp.int32),
            jax.ShapeDtypeStruct((padn,), jnp.float32),
        ],
        mesh=mesh,
        scratch_types=[
            pltpu.VMEM((nk,), jnp.int32),
            pltpu.VMEM((nk,), jnp.int32),
            pltpu.VMEM((nk,), jnp.float32),
            pltpu.VMEM((padn,), jnp.int32),
            pltpu.VMEM((padn,), jnp.float32),
        ],
        compiler_params=pltpu.CompilerParams(needs_layout_passes=False),
    )
    def k(p_hbm, tok_hbm, g_hbm, dflt_hbm, st_hbm, gs_hbm, p_v, t_v, g_v, st_v, gs_v):
        cid = lax.axis_index("c")
        sid = lax.axis_index("s")

        @pl.when((cid == 0) & (sid == 0))
        def _():
            pltpu.sync_copy(p_hbm, p_v)
            pltpu.sync_copy(tok_hbm, t_v)
            pltpu.sync_copy(g_hbm, g_v)
            pltpu.sync_copy(dflt_hbm, st_v)
            zf = jnp.zeros((16,), jnp.float32)

            def memset(i, _):
                gs_v[pl.ds(i * 16, 16)] = zf
                return 0

            lax.fori_loop(0, padn // 16, memset, 0)

            def scat(j, _):
                idx = p_v[pl.ds(j * 16, 16)]
                plsc.store_scatter(st_v, [idx], t_v[pl.ds(j * 16, 16)])
                plsc.store_scatter(gs_v, [idx], g_v[pl.ds(j * 16, 16)])
                return 0

            lax.fori_loop(0, nk // 16, scat, 0)
            pltpu.sync_copy(st_v, st_hbm)
            pltpu.sync_copy(gs_v, gs_hbm)

    return k(p_flat, tok_flat, g_flat, dflt)


def _sc_gather_rows(table, idx, chunk):
    rows = idx.shape[0]
    d = table.shape[1]
    nw = 32
    per_w = rows // nw
    nch = per_w // chunk
    mesh = plsc.VectorSubcoreMesh(core_axis_name="c", subcore_axis_name="s")

    @functools.partial(
        pl.kernel,
        out_type=jax.ShapeDtypeStruct((rows, d), table.dtype),
        mesh=mesh,
        scratch_types=[
            pltpu.VMEM((chunk,), jnp.int32),
            pltpu.VMEM((chunk, d), table.dtype),
            pltpu.SemaphoreType.DMA,
        ],
    )
    def k(tab_hbm, idx_hbm, out_hbm, idx_v, rows_v, sem):
        wid = lax.axis_index("s") * 2 + lax.axis_index("c")

        def body(ch, _):
            base = wid * per_w + ch * chunk
            pltpu.sync_copy(idx_hbm.at[pl.ds(base, chunk)], idx_v)
            pltpu.async_copy(tab_hbm.at[idx_v], rows_v, sem).wait()
            pltpu.sync_copy(rows_v, out_hbm.at[pl.ds(base, chunk)])
            return 0

        lax.fori_loop(0, nch, body, 0)

    return k(table, idx)


def _shared_body(x_ref, wg_ref, wu_ref, wd_ref, out_ref, acc_ref,
                 wgb_ref, wub_ref, wdb_ref, *, nf, tb):
    f = pl.program_id(0)
    t = pl.program_id(1)

    @pl.when(t == 0)
    def _():
        wgb_ref[...] = wg_ref[...].astype(jnp.bfloat16)
        wub_ref[...] = wu_ref[...].astype(jnp.bfloat16)
        wdb_ref[...] = wd_ref[...].astype(jnp.bfloat16)

    x = x_ref[...].astype(jnp.bfloat16)
    g = lax.dot_general(
        x, wgb_ref[...], (((1,), (1,)), ((), ())),
        preferred_element_type=jnp.float32,
    ).astype(jnp.bfloat16)
    u = lax.dot_general(
        x, wub_ref[...], (((1,), (1,)), ((), ())),
        preferred_element_type=jnp.float32,
    ).astype(jnp.bfloat16)
    h = g * jax.nn.sigmoid(g) * u
    part = lax.dot_general(
        h, wdb_ref[...], (((1,), (1,)), ((), ())),
        preferred_element_type=jnp.float32,
    )
    rows = pl.ds(t * tb, tb)

    @pl.when(f == 0)
    def _():
        acc_ref[rows, :] = part

    @pl.when(f != 0)
    def _():
        acc_ref[rows, :] += part

    @pl.when(f == nf - 1)
    def _():
        out_ref[...] = acc_ref[rows, :]


def _grouped_body(be_ref, xs_ref, gs_ref, wg_ref, wu_ref, wd_ref, y_ref,
                  acc_ref, wgb_ref, wub_ref, wdb_ref, *, nf, tb, bh, ne):
    h_i = pl.program_id(0)
    f = pl.program_id(1)
    b = pl.program_id(2)
    gb = h_i * bh + b
    active = be_ref[gb] < ne
    fresh = (b == 0) | (be_ref[gb] != be_ref[jnp.maximum(gb - 1, 0)])

    @pl.when(active)
    def _():
        @pl.when(fresh)
        def _():
            wgb_ref[...] = wg_ref[0].astype(jnp.bfloat16)
            wub_ref[...] = wu_ref[0].astype(jnp.bfloat16)
            wdb_ref[...] = wd_ref[0].astype(jnp.bfloat16)

        x = xs_ref[...].astype(jnp.bfloat16)
        g = lax.dot_general(
            x, wgb_ref[...], (((1,), (1,)), ((), ())),
            preferred_element_type=jnp.float32,
        ).astype(jnp.bfloat16)
        u = lax.dot_general(
            x, wub_ref[...], (((1,), (1,)), ((), ())),
            preferred_element_type=jnp.float32,
        ).astype(jnp.bfloat16)
        hh = g * jax.nn.sigmoid(g) * u
        part = lax.dot_general(
            hh, wdb_ref[...], (((1,), (1,)), ((), ())),
            preferred_element_type=jnp.float32,
        )
        rows = pl.ds(b * tb, tb)

        @pl.when(f == 0)
        def _():
            acc_ref[rows, :] = part

        @pl.when(f != 0)
        def _():
            acc_ref[rows, :] += part

        @pl.when(f == nf - 1)
        def _():
            y_ref[...] = gs_ref[...] * acc_ref[rows, :]


def _sc_gather_flat(table, idx, chunk):
    rows = idx.shape[0]
    d = table.shape[1]
    nw = 32
    per_w = rows // nw
    nch = per_w // chunk
    mesh = plsc.VectorSubcoreMesh(core_axis_name="c", subcore_axis_name="s")

    @functools.partial(
        pl.kernel,
        out_type=jax.ShapeDtypeStruct((rows, d), table.dtype),
        mesh=mesh,
        scratch_types=[
            pltpu.VMEM((per_w,), jnp.int32),
            pltpu.VMEM((chunk, d), table.dtype),
            pltpu.VMEM((chunk, d), table.dtype),
            pltpu.SemaphoreType.DMA,
            pltpu.SemaphoreType.DMA,
            pltpu.SemaphoreType.DMA,
            pltpu.SemaphoreType.DMA,
        ],
    )
    def k(tab_hbm, idx_hbm, out_hbm, i_v, b0, b1, g0, g1, w0, w1):
        wid = lax.axis_index("s") * 2 + lax.axis_index("c")
        base = wid * per_w
        pltpu.sync_copy(idx_hbm.at[pl.ds(base, per_w)], i_v)
        bufs = (b0, b1)
        gsem = (g0, g1)
        wsem = (w0, w1)
        writes = [None, None]
        for ch in range(nch):
            s = ch % 2
            if writes[s] is not None:
                writes[s].wait()
            pltpu.async_copy(
                tab_hbm.at[i_v.at[pl.ds(ch * chunk, chunk)]], bufs[s], gsem[s]
            ).wait()
            writes[s] = pltpu.async_copy(
                bufs[s], out_hbm.at[pl.ds(base + ch * chunk, chunk)], wsem[s]
            )
        for wr in writes:
            if wr is not None:
                wr.wait()

    return k(table, idx)


def _combine_body(sh_ref, yg_ref, out_ref, *, d):
    out_ref[...] = sh_ref[...] + yg_ref[:, :d] + yg_ref[:, d:]


def kernel(X, W_router, expert_bias, Wg_s, Wu_s, Wd_s, Wg_e, Wu_e, Wd_e):
    batch, seq, d = X.shape
    n = batch * seq
    ne, dff, _ = Wg_e.shape
    xf = X.reshape(n, d)

    tb = min(256, n)
    nt = n // tb
    fb = min(1024, dff)
    nf = dff // fb
    nblk = (n * 2) // tb + ne
    padn = nblk * tb
    nh = 2
    bh = nblk // nh

    topw, tope, r, cnt3 = pl.pallas_call(
        functools.partial(_router_body, n_exp=ne, tb=tb),
        grid=(nt,),
        in_specs=[
            pl.BlockSpec((tb, d), lambda t: (t, 0)),
            pl.BlockSpec((ne, d), lambda t: (0, 0)),
            pl.BlockSpec((1, ne), lambda t: (0, 0)),
        ],
        out_specs=[
            pl.BlockSpec((tb, 2), lambda t: (t, 0)),
            pl.BlockSpec((tb, 2), lambda t: (t, 0)),
            pl.BlockSpec((tb, 2), lambda t: (t, 0)),
            pl.BlockSpec((1, 1, ne), lambda t: (t, 0, 0)),
        ],
        out_shape=[
            jax.ShapeDtypeStruct((n, 2), jnp.float32),
            jax.ShapeDtypeStruct((n, 2), jnp.int32),
            jax.ShapeDtypeStruct((n, 2), jnp.int32),
            jax.ShapeDtypeStruct((nt, 1, ne), jnp.float32),
        ],
    )(xf, W_router, expert_bias.reshape(1, ne))

    p, be3 = pl.pallas_call(
        functools.partial(_dispatch_body, n_exp=ne, tb=tb, nt=nt, nblk=nblk),
        grid=(nt,),
        in_specs=[
            pl.BlockSpec((nt, 1, ne), lambda t: (0, 0, 0)),
            pl.BlockSpec((tb, 2), lambda t: (t, 0)),
            pl.BlockSpec((tb, 2), lambda t: (t, 0)),
        ],
        out_specs=[
            pl.BlockSpec((tb, 2), lambda t: (t, 0)),
            pl.BlockSpec((1, 1, nblk), lambda t: (0, 0, 0)),
        ],
        out_shape=[
            jax.ShapeDtypeStruct((n, 2), jnp.int32),
            jax.ShapeDtypeStruct((1, 1, nblk), jnp.int32),
        ],
    )(cnt3, tope, r)

    tok_flat = jnp.arange(n * 2, dtype=jnp.int32) // 2
    dflt = jnp.arange(padn, dtype=jnp.int32) % n
    sorted_token, gates_sorted = _sc_permute(
        p.reshape(n * 2), tok_flat, topw.reshape(n * 2), dflt, padn
    )

    x_sorted = _sc_gather_flat(xf, sorted_token, chunk=40)

    shared = pl.pallas_call(
        functools.partial(_shared_body, nf=nf, tb=tb),
        grid=(nf, nt),
        in_specs=[
            pl.BlockSpec((tb, d), lambda f, t: (t, 0)),
            pl.BlockSpec((fb, d), lambda f, t: (f, 0)),
            pl.BlockSpec((fb, d), lambda f, t: (f, 0)),
            pl.BlockSpec((d, fb), lambda f, t: (0, f)),
        ],
        out_specs=pl.BlockSpec(
            (tb, d), lambda f, t: (jnp.where(f == nf - 1, t, 0), 0)
        ),
        out_shape=jax.ShapeDtypeStruct((n, d), jnp.float32),
        scratch_shapes=[
            pltpu.VMEM((n, d), jnp.float32),
            pltpu.VMEM((fb, d), jnp.bfloat16),
            pltpu.VMEM((fb, d), jnp.bfloat16),
            pltpu.VMEM((d, fb), jnp.bfloat16),
        ],
    )(xf, Wg_s, Wu_s, Wd_s)

    y = pl.pallas_call(
        functools.partial(_grouped_body, nf=nf, tb=tb, bh=bh, ne=ne),
        grid_spec=pltpu.PrefetchScalarGridSpec(
            num_scalar_prefetch=1,
            grid=(nh, nf, bh),
            in_specs=[
                pl.BlockSpec((tb, d), lambda h, f, b, be: (h * bh + b, 0)),
                pl.BlockSpec((tb, 1), lambda h, f, b, be: (h * bh + b, 0)),
                pl.BlockSpec((1, fb, d),
                             lambda h, f, b, be: (jnp.minimum(be[h * bh + b], 7), f, 0)),
                pl.BlockSpec((1, fb, d),
                             lambda h, f, b, be: (jnp.minimum(be[h * bh + b], 7), f, 0)),
                pl.BlockSpec((1, d, fb),
                             lambda h, f, b, be: (jnp.minimum(be[h * bh + b], 7), 0, f)),
            ],
            out_specs=pl.BlockSpec(
                (tb, d),
                lambda h, f, b, be: (jnp.where(f == nf - 1, h * bh + b, h * bh), 0),
            ),
            scratch_shapes=[
                pltpu.VMEM((bh * tb, d), jnp.float32),
                pltpu.VMEM((fb, d), jnp.bfloat16),
                pltpu.VMEM((fb, d), jnp.bfloat16),
                pltpu.VMEM((d, fb), jnp.bfloat16),
            ],
        ),
        out_shape=jax.ShapeDtypeStruct((padn, d), jnp.float32),
    )(be3.reshape(nblk), x_sorted, gates_sorted.reshape(padn, 1),
      Wg_e, Wu_e, Wd_e)

    yg = _sc_gather_flat(y, p.reshape(n * 2), chunk=32)
    yg2 = yg.reshape(n, 2 * d)

    out = pl.pallas_call(
        functools.partial(_combine_body, d=d),
        grid=(nt,),
        in_specs=[
            pl.BlockSpec((tb, d), lambda t: (t, 0)),
            pl.BlockSpec((tb, 2 * d), lambda t: (t, 0)),
        ],
        out_specs=pl.BlockSpec((tb, d), lambda t: (t, 0)),
        out_shape=jax.ShapeDtypeStruct((n, d), jnp.float32),
    )(shared, yg2)

    return out.reshape(batch, seq, d)


# speedup vs baseline: 1.0503x; 1.0503x over previous
import functools

import jax
import jax.numpy as jnp
from jax import lax
from jax.experimental import pallas as pl
from jax.experimental.pallas import tpu as pltpu
from jax.experimental.pallas import tpu_sc as plsc


def _router_body(x_ref, wr_ref, b_ref, topw_ref, tope_ref, r_ref, cnt_ref,
                 *, n_exp, tb):
    x = x_ref[...]
    wr = wr_ref[...]
    logits = lax.dot_general(
        x, wr, (((1,), (1,)), ((), ())), preferred_element_type=jnp.float32
    ) + b_ref[...]
    w = jax.nn.softmax(logits, axis=-1)
    lane = lax.broadcasted_iota(jnp.int32, w.shape, 1)
    rank = jnp.zeros_like(w)
    for j in range(n_exp):
        wj = w[:, j : j + 1]
        rank += (wj > w).astype(jnp.float32)
        rank += ((wj == w) & (j < lane)).astype(jnp.float32)
    on0 = rank < 0.5
    on1 = (rank >= 0.5) & (rank < 1.5)
    ew = jnp.exp(w)
    denom = jnp.sum(jnp.where(on0 | on1, ew, 0.0), axis=1, keepdims=True)
    w0 = jnp.sum(jnp.where(on0, ew, 0.0), axis=1, keepdims=True) / denom
    w1 = jnp.sum(jnp.where(on1, ew, 0.0), axis=1, keepdims=True) / denom
    e0 = jnp.sum(jnp.where(on0, lane, 0), axis=1, keepdims=True)
    e1 = jnp.sum(jnp.where(on1, lane, 0), axis=1, keepdims=True)
    topw_ref[...] = jnp.concatenate([w0, w1], axis=1)
    tope_ref[...] = jnp.concatenate([e0, e1], axis=1)
    occ = (on0 | on1).astype(jnp.bfloat16)
    row_i = lax.broadcasted_iota(jnp.int32, (tb, tb), 0)
    col_i = lax.broadcasted_iota(jnp.int32, (tb, tb), 1)
    ltr = (row_i > col_i).astype(jnp.bfloat16)
    cum = lax.dot_general(
        ltr, occ, (((1,), (0,)), ((), ())), preferred_element_type=jnp.float32
    )
    r0 = jnp.sum(jnp.where(on0, cum, 0.0), axis=1, keepdims=True)
    r1 = jnp.sum(jnp.where(on1, cum, 0.0), axis=1, keepdims=True)
    r_ref[...] = jnp.concatenate([r0, r1], axis=1).astype(jnp.int32)
    cnt_ref[...] = jnp.sum(occ.astype(jnp.float32), axis=0, keepdims=True)[None]


def _dispatch_body(cnt_ref, tope_ref, r_ref, p_ref, be_ref,
                   *, n_exp, tb, nt, nblk):
    t = pl.program_id(0)
    cnt = cnt_ref[:, 0, :]
    c = jnp.sum(cnt, axis=0, keepdims=True)
    row = lax.broadcasted_iota(jnp.int32, cnt.shape, 0)
    base = jnp.sum(jnp.where(row < t, cnt, 0.0), axis=0, keepdims=True)
    pb = jnp.floor((c + (tb - 1)) / tb)
    lane = lax.broadcasted_iota(jnp.int32, (1, n_exp), 1)
    start = jnp.zeros((1, n_exp), jnp.float32)
    for e in range(n_exp - 1):
        start += jnp.where(lane > e, pb[0:1, e : e + 1] * tb, 0.0)
    sb = start + base
    tope = tope_ref[...]
    r = r_ref[...].astype(jnp.float32)
    lane_tb = lax.broadcasted_iota(jnp.int32, (tope.shape[0], n_exp), 1)
    ps = []
    for k in range(2):
        onek = tope[:, k : k + 1] == lane_tb
        pk = jnp.sum(jnp.where(onek, sb, 0.0), axis=1, keepdims=True)
        ps.append(pk + r[:, k : k + 1])
    p_ref[...] = jnp.concatenate(ps, axis=1).astype(jnp.int32)

    @pl.when(t == 0)
    def _():
        lane_b = lax.broadcasted_iota(jnp.int32, (1, nblk), 1)
        acc = jnp.zeros((1, nblk), jnp.int32)
        run = pb[0:1, 0:1]
        for e in range(1, n_exp + 1):
            acc += (lane_b >= run.astype(jnp.int32)).astype(jnp.int32)
            if e < n_exp:
                run = run + pb[0:1, e : e + 1]
        be_ref[...] = acc[None]


def _sc_permute(p_flat, tok_flat, g_flat, dflt, padn):
    nk = p_flat.shape[0]
    mesh = plsc.VectorSubcoreMesh(core_axis_name="c", subcore_axis_name="s")

    @functools.partial(
        pl.kernel,
        out_type=[
            jax.ShapeDtypeStruct((padn,), jnp.int32),
            jax.ShapeDtypeStruct((padn,), jnp.float32),
        ],
        mesh=mesh,
        scratch_types=[
            pltpu.VMEM((nk,), jnp.int32),
            pltpu.VMEM((nk,), jnp.int32),
            pltpu.VMEM((nk,), jnp.float32),
            pltpu.VMEM((padn,), jnp.int32),
            pltpu.VMEM((padn,), jnp.float32),
        ],
        compiler_params=pltpu.CompilerParams(needs_layout_passes=False),
    )
    def k(p_hbm, tok_hbm, g_hbm, dflt_hbm, st_hbm, gs_hbm, p_v, t_v, g_v, st_v, gs_v):
        cid = lax.axis_index("c")
        sid = lax.axis_index("s")

        @pl.when((cid == 0) & (sid == 0))
        def _():
            pltpu.sync_copy(p_hbm, p_v)
            pltpu.sync_copy(tok_hbm, t_v)
            pltpu.sync_copy(g_hbm, g_v)
            pltpu.sync_copy(dflt_hbm, st_v)
            zf = jnp.zeros((16,), jnp.float32)

            def memset(i, _):
                gs_v[pl.ds(i * 16, 16)] = zf
                return 0

            lax.fori_loop(0, padn // 16, memset, 0)

            def scat(j, _):
                idx = p_v[pl.ds(j * 16, 16)]
                plsc.store_scatter(st_v, [idx], t_v[pl.ds(j * 16, 16)])
                plsc.store_scatter(gs_v, [idx], g_v[pl.ds(j * 16, 16)])
                return 0

            lax.fori_loop(0, nk // 16, scat, 0)
            pltpu.sync_copy(st_v, st_hbm)
            pltpu.sync_copy(gs_v, gs_hbm)

    return k(p_flat, tok_flat, g_flat, dflt)


def _sc_gather_rows(table, idx, chunk):
    rows = idx.shape[0]
    d = table.shape[1]
    nw = 32
    per_w = rows // nw
    nch = per_w // chunk
    mesh = plsc.VectorSubcoreMesh(core_axis_name="c", subcore_axis_name="s")

    @functools.partial(
        pl.kernel,
        out_type=jax.ShapeDtypeStruct((rows, d), table.dtype),
        mesh=mesh,
        scratch_types=[
            pltpu.VMEM((chunk,), jnp.int32),
            pltpu.VMEM((chunk, d), table.dtype),
            pltpu.SemaphoreType.DMA,
        ],
    )
    def k(tab_hbm, idx_hbm, out_hbm, idx_v, rows_v, sem):
        wid = lax.axis_index("s") * 2 + lax.axis_index("c")

        def body(ch, _):
            base = wid * per_w + ch * chunk
            pltpu.sync_copy(idx_hbm.at[pl.ds(base, chunk)], idx_v)
            pltpu.async_copy(tab_hbm.at[idx_v], rows_v, sem).wait()
            pltpu.sync_copy(rows_v, out_hbm.at[pl.ds(base, chunk)])
            return 0

        lax.fori_loop(0, nch, body, 0)

    return k(table, idx)


def _shared_body(x_ref, wg_ref, wu_ref, wd_ref, out_ref, acc_ref,
                 wgb_ref, wub_ref, wdb_ref, *, nf, tb):
    f = pl.program_id(0)
    t = pl.program_id(1)

    @pl.when(t == 0)
    def _():
        wgb_ref[...] = wg_ref[...].astype(jnp.bfloat16)
        wub_ref[...] = wu_ref[...].astype(jnp.bfloat16)
        wdb_ref[...] = wd_ref[...].astype(jnp.bfloat16)

    x = x_ref[...].astype(jnp.bfloat16)
    g = lax.dot_general(
        x, wgb_ref[...], (((1,), (1,)), ((), ())),
        preferred_element_type=jnp.float32,
    ).astype(jnp.bfloat16)
    u = lax.dot_general(
        x, wub_ref[...], (((1,), (1,)), ((), ())),
        preferred_element_type=jnp.float32,
    ).astype(jnp.bfloat16)
    h = g * jax.nn.sigmoid(g) * u
    part = lax.dot_general(
        h, wdb_ref[...], (((1,), (1,)), ((), ())),
        preferred_element_type=jnp.float32,
    )
    rows = pl.ds(t * tb, tb)

    @pl.when(f == 0)
    def _():
        acc_ref[rows, :] = part

    @pl.when(f != 0)
    def _():
        acc_ref[rows, :] += part

    @pl.when(f == nf - 1)
    def _():
        out_ref[...] = acc_ref[rows, :]


def _grouped_body(be_ref, xs_ref, gs_ref, wg_ref, wu_ref, wd_ref, y_ref,
                  acc_ref, wgb_ref, wub_ref, wdb_ref, *, nf, tb, bh, ne):
    h_i = pl.program_id(0)
    f = pl.program_id(1)
    b = pl.program_id(2)
    gb = h_i * bh + b
    active = be_ref[gb] < ne
    fresh = (b == 0) | (be_ref[gb] != be_ref[jnp.maximum(gb - 1, 0)])

    @pl.when(active)
    def _():
        @pl.when(fresh)
        def _():
            wgb_ref[...] = wg_ref[0].astype(jnp.bfloat16)
            wub_ref[...] = wu_ref[0].astype(jnp.bfloat16)
            wdb_ref[...] = wd_ref[0].astype(jnp.bfloat16)

        x = xs_ref[...].astype(jnp.bfloat16)
        g = lax.dot_general(
            x, wgb_ref[...], (((1,), (1,)), ((), ())),
            preferred_element_type=jnp.float32,
        ).astype(jnp.bfloat16)
        u = lax.dot_general(
            x, wub_ref[...], (((1,), (1,)), ((), ())),
            preferred_element_type=jnp.float32,
        ).astype(jnp.bfloat16)
        hh = g * jax.nn.sigmoid(g) * u
        part = lax.dot_general(
            hh, wdb_ref[...], (((1,), (1,)), ((), ())),
            preferred_element_type=jnp.float32,
        )
        rows = pl.ds(b * tb, tb)

        @pl.when(f == 0)
        def _():
            acc_ref[rows, :] = part

        @pl.when(f != 0)
        def _():
            acc_ref[rows, :] += part

        @pl.when(f == nf - 1)
        def _():
            y_ref[...] = gs_ref[...] * acc_ref[rows, :]


def _sc_gather_flat(table, idx, chunk):
    rows = idx.shape[0]
    d = table.shape[1]
    nw = 32
    per_w = rows // nw
    nch = per_w // chunk
    mesh = plsc.VectorSubcoreMesh(core_axis_name="c", subcore_axis_name="s")

    @functools.partial(
        pl.kernel,
        out_type=jax.ShapeDtypeStruct((rows, d), table.dtype),
        mesh=mesh,
        scratch_types=[
            pltpu.VMEM((per_w,), jnp.int32),
            pltpu.VMEM((chunk, d), table.dtype),
            pltpu.VMEM((chunk, d), table.dtype),
            pltpu.SemaphoreType.DMA,
            pltpu.SemaphoreType.DMA,
            pltpu.SemaphoreType.DMA,
            pltpu.SemaphoreType.DMA,
        ],
    )
    def k(tab_hbm, idx_hbm, out_hbm, i_v, b0, b1, g0, g1, w0, w1):
        wid = lax.axis_index("s") * 2 + lax.axis_index("c")
        base = wid * per_w
        pltpu.sync_copy(idx_hbm.at[pl.ds(base, per_w)], i_v)
        bufs = (b0, b1)
        gsem = (g0, g1)
        wsem = (w0, w1)
        writes = [None, None]
        for ch in range(nch):
            s = ch % 2
            if writes[s] is not None:
                writes[s].wait()
            pltpu.async_copy(
                tab_hbm.at[i_v.at[pl.ds(ch * chunk, chunk)]], bufs[s], gsem[s]
            ).wait()
            writes[s] = pltpu.async_copy(
                bufs[s], out_hbm.at[pl.ds(base + ch * chunk, chunk)], wsem[s]
            )
        for wr in writes:
            if wr is not None:
                wr.wait()

    return k(table, idx)


def _combine_body(sh_ref, y0_ref, y1_ref, out_ref):
    out_ref[...] = sh_ref[...] + y0_ref[...] + y1_ref[...]


def kernel(X, W_router, expert_bias, Wg_s, Wu_s, Wd_s, Wg_e, Wu_e, Wd_e):
    batch, seq, d = X.shape
    n = batch * seq
    ne, dff, _ = Wg_e.shape
    xf = X.reshape(n, d)

    tb = min(256, n)
    nt = n // tb
    fb = min(1024, dff)
    nf = dff // fb
    nblk = (n * 2) // tb + ne
    padn = nblk * tb
    nh = 2
    bh = nblk // nh

    topw, tope, r, cnt3 = pl.pallas_call(
        functools.partial(_router_body, n_exp=ne, tb=tb),
        grid=(nt,),
        in_specs=[
            pl.BlockSpec((tb, d), lambda t: (t, 0)),
            pl.BlockSpec((ne, d), lambda t: (0, 0)),
            pl.BlockSpec((1, ne), lambda t: (0, 0)),
        ],
        out_specs=[
            pl.BlockSpec((tb, 2), lambda t: (t, 0)),
            pl.BlockSpec((tb, 2), lambda t: (t, 0)),
            pl.BlockSpec((tb, 2), lambda t: (t, 0)),
            pl.BlockSpec((1, 1, ne), lambda t: (t, 0, 0)),
        ],
        out_shape=[
            jax.ShapeDtypeStruct((n, 2), jnp.float32),
            jax.ShapeDtypeStruct((n, 2), jnp.int32),
            jax.ShapeDtypeStruct((n, 2), jnp.int32),
            jax.ShapeDtypeStruct((nt, 1, ne), jnp.float32),
        ],
    )(xf, W_router, expert_bias.reshape(1, ne))

    p, be3 = pl.pallas_call(
        functools.partial(_dispatch_body, n_exp=ne, tb=tb, nt=nt, nblk=nblk),
        grid=(nt,),
        in_specs=[
            pl.BlockSpec((nt, 1, ne), lambda t: (0, 0, 0)),
            pl.BlockSpec((tb, 2), lambda t: (t, 0)),
            pl.BlockSpec((tb, 2), lambda t: (t, 0)),
        ],
        out_specs=[
            pl.BlockSpec((tb, 2), lambda t: (t, 0)),
            pl.BlockSpec((1, 1, nblk), lambda t: (0, 0, 0)),
        ],
        out_shape=[
            jax.ShapeDtypeStruct((n, 2), jnp.int32),
            jax.ShapeDtypeStruct((1, 1, nblk), jnp.int32),
        ],
    )(cnt3, tope, r)

    tok_flat = jnp.arange(n * 2, dtype=jnp.int32) // 2
    dflt = jnp.arange(padn, dtype=jnp.int32) % n
    sorted_token, gates_sorted = _sc_permute(
        p.reshape(n * 2), tok_flat, topw.reshape(n * 2), dflt, padn
    )

    x_sorted = _sc_gather_flat(xf, sorted_token, chunk=40)

    shared = pl.pallas_call(
        functools.partial(_shared_body, nf=nf, tb=tb),
        grid=(nf, nt),
        in_specs=[
            pl.BlockSpec((tb, d), lambda f, t: (t, 0)),
            pl.BlockSpec((fb, d), lambda f, t: (f, 0)),
            pl.BlockSpec((fb, d), lambda f, t: (f, 0)),
            pl.BlockSpec((d, fb), lambda f, t: (0, f)),
        ],
        out_specs=pl.BlockSpec(
            (tb, d), lambda f, t: (jnp.where(f == nf - 1, t, 0), 0)
        ),
        out_shape=jax.ShapeDtypeStruct((n, d), jnp.float32),
        scratch_shapes=[
            pltpu.VMEM((n, d), jnp.float32),
            pltpu.VMEM((fb, d), jnp.bfloat16),
            pltpu.VMEM((fb, d), jnp.bfloat16),
            pltpu.VMEM((d, fb), jnp.bfloat16),
        ],
    )(xf, Wg_s, Wu_s, Wd_s)

    y = pl.pallas_call(
        functools.partial(_grouped_body, nf=nf, tb=tb, bh=bh, ne=ne),
        grid_spec=pltpu.PrefetchScalarGridSpec(
            num_scalar_prefetch=1,
            grid=(nh, nf, bh),
            in_specs=[
                pl.BlockSpec((tb, d), lambda h, f, b, be: (h * bh + b, 0)),
                pl.BlockSpec((tb, 1), lambda h, f, b, be: (h * bh + b, 0)),
                pl.BlockSpec((1, fb, d),
                             lambda h, f, b, be: (jnp.minimum(be[h * bh + b], 7), f, 0)),
                pl.BlockSpec((1, fb, d),
                             lambda h, f, b, be: (jnp.minimum(be[h * bh + b], 7), f, 0)),
                pl.BlockSpec((1, d, fb),
                             lambda h, f, b, be: (jnp.minimum(be[h * bh + b], 7), 0, f)),
            ],
            out_specs=pl.BlockSpec(
                (tb, d),
                lambda h, f, b, be: (jnp.where(f == nf - 1, h * bh + b, h * bh), 0),
            ),
            scratch_shapes=[
                pltpu.VMEM((bh * tb, d), jnp.float32),
                pltpu.VMEM((fb, d), jnp.bfloat16),
                pltpu.VMEM((fb, d), jnp.bfloat16),
                pltpu.VMEM((d, fb), jnp.bfloat16),
            ],
        ),
        out_shape=jax.ShapeDtypeStruct((padn, d), jnp.float32),
    )(be3.reshape(nblk), x_sorted, gates_sorted.reshape(padn, 1),
      Wg_e, Wu_e, Wd_e)

    y0 = _sc_gather_flat(y, p[:, 0], chunk=32)
    y1 = _sc_gather_flat(y, p[:, 1], chunk=32)

    out = pl.pallas_call(
        _combine_body,
        grid=(nt,),
        in_specs=[
            pl.BlockSpec((tb, d), lambda t: (t, 0)),
            pl.BlockSpec((tb, d), lambda t: (t, 0)),
            pl.BlockSpec((tb, d), lambda t: (t, 0)),
        ],
        out_specs=pl.BlockSpec((tb, d), lambda t: (t, 0)),
        out_shape=jax.ShapeDtypeStruct((n, d), jnp.float32),
    )(shared, y0, y1)

    return out.reshape(batch, seq, d)


# speedup vs baseline: 1.1108x; 1.0576x over previous
import functools

import jax
import jax.numpy as jnp
from jax import lax
from jax.experimental import pallas as pl
from jax.experimental.pallas import tpu as pltpu
from jax.experimental.pallas import tpu_sc as plsc


def _router_body(x_ref, wr_ref, b_ref, topw_ref, tope_ref, r_ref, cnt_ref,
                 *, n_exp, tb):
    x = x_ref[...]
    wr = wr_ref[...]
    logits = lax.dot_general(
        x, wr, (((1,), (1,)), ((), ())), preferred_element_type=jnp.float32
    ) + b_ref[...]
    w = jax.nn.softmax(logits, axis=-1)
    lane = lax.broadcasted_iota(jnp.int32, w.shape, 1)
    rank = jnp.zeros_like(w)
    for j in range(n_exp):
        wj = w[:, j : j + 1]
        rank += (wj > w).astype(jnp.float32)
        rank += ((wj == w) & (j < lane)).astype(jnp.float32)
    on0 = rank < 0.5
    on1 = (rank >= 0.5) & (rank < 1.5)
    ew = jnp.exp(w)
    denom = jnp.sum(jnp.where(on0 | on1, ew, 0.0), axis=1, keepdims=True)
    w0 = jnp.sum(jnp.where(on0, ew, 0.0), axis=1, keepdims=True) / denom
    w1 = jnp.sum(jnp.where(on1, ew, 0.0), axis=1, keepdims=True) / denom
    e0 = jnp.sum(jnp.where(on0, lane, 0), axis=1, keepdims=True)
    e1 = jnp.sum(jnp.where(on1, lane, 0), axis=1, keepdims=True)
    topw_ref[...] = jnp.concatenate([w0, w1], axis=1)
    tope_ref[...] = jnp.concatenate([e0, e1], axis=1)
    occ = (on0 | on1).astype(jnp.bfloat16)
    row_i = lax.broadcasted_iota(jnp.int32, (tb, tb), 0)
    col_i = lax.broadcasted_iota(jnp.int32, (tb, tb), 1)
    ltr = (row_i > col_i).astype(jnp.bfloat16)
    cum = lax.dot_general(
        ltr, occ, (((1,), (0,)), ((), ())), preferred_element_type=jnp.float32
    )
    r0 = jnp.sum(jnp.where(on0, cum, 0.0), axis=1, keepdims=True)
    r1 = jnp.sum(jnp.where(on1, cum, 0.0), axis=1, keepdims=True)
    r_ref[...] = jnp.concatenate([r0, r1], axis=1).astype(jnp.int32)
    cnt_ref[...] = jnp.sum(occ.astype(jnp.float32), axis=0, keepdims=True)[None]


def _dispatch_body(cnt_ref, tope_ref, r_ref, p_ref, be_ref,
                   *, n_exp, tb, nt, nblk):
    t = pl.program_id(0)
    cnt = cnt_ref[:, 0, :]
    c = jnp.sum(cnt, axis=0, keepdims=True)
    row = lax.broadcasted_iota(jnp.int32, cnt.shape, 0)
    base = jnp.sum(jnp.where(row < t, cnt, 0.0), axis=0, keepdims=True)
    pb = jnp.floor((c + (tb - 1)) / tb)
    lane = lax.broadcasted_iota(jnp.int32, (1, n_exp), 1)
    start = jnp.zeros((1, n_exp), jnp.float32)
    for e in range(n_exp - 1):
        start += jnp.where(lane > e, pb[0:1, e : e + 1] * tb, 0.0)
    sb = start + base
    tope = tope_ref[...]
    r = r_ref[...].astype(jnp.float32)
    lane_tb = lax.broadcasted_iota(jnp.int32, (tope.shape[0], n_exp), 1)
    ps = []
    for k in range(2):
        onek = tope[:, k : k + 1] == lane_tb
        pk = jnp.sum(jnp.where(onek, sb, 0.0), axis=1, keepdims=True)
        ps.append(pk + r[:, k : k + 1])
    p_ref[...] = jnp.concatenate(ps, axis=1).astype(jnp.int32)

    @pl.when(t == 0)
    def _():
        lane_b = lax.broadcasted_iota(jnp.int32, (1, nblk), 1)
        acc = jnp.zeros((1, nblk), jnp.int32)
        run = pb[0:1, 0:1]
        for e in range(1, n_exp + 1):
            acc += (lane_b >= run.astype(jnp.int32)).astype(jnp.int32)
            if e < n_exp:
                run = run + pb[0:1, e : e + 1]
        be_ref[...] = acc[None]


def _sc_permute(p_flat, tok_flat, g_flat, dflt, padn):
    nk = p_flat.shape[0]
    mesh = plsc.VectorSubcoreMesh(core_axis_name="c", subcore_axis_name="s")

    @functools.partial(
        pl.kernel,
        out_type=[
            jax.ShapeDtypeStruct((padn,), jnp.int32),
            jax.ShapeDtypeStruct((padn,), jnp.float32),
        ],
        mesh=mesh,
        scratch_types=[
            pltpu.VMEM((nk,), jnp.int32),
            pltpu.VMEM((nk,), jnp.int32),
            pltpu.VMEM((nk,), jnp.float32),
            pltpu.VMEM((padn,), jnp.int32),
            pltpu.VMEM((padn,), jnp.float32),
        ],
        compiler_params=pltpu.CompilerParams(needs_layout_passes=False),
    )
    def k(p_hbm, tok_hbm, g_hbm, dflt_hbm, st_hbm, gs_hbm, p_v, t_v, g_v, st_v, gs_v):
        cid = lax.axis_index("c")
        sid = lax.axis_index("s")

        @pl.when((cid == 0) & (sid == 0))
        def _():
            pltpu.sync_copy(p_hbm, p_v)
            pltpu.sync_copy(tok_hbm, t_v)
            pltpu.sync_copy(g_hbm, g_v)
            pltpu.sync_copy(dflt_hbm, st_v)
            zf = jnp.zeros((16,), jnp.float32)

            def memset(i, _):
                gs_v[pl.ds(i * 16, 16)] = zf
                return 0

            lax.fori_loop(0, padn // 16, memset, 0)

            def scat(j, _):
                idx = p_v[pl.ds(j * 16, 16)]
                plsc.store_scatter(st_v, [idx], t_v[pl.ds(j * 16, 16)])
                plsc.store_scatter(gs_v, [idx], g_v[pl.ds(j * 16, 16)])
                return 0

            lax.fori_loop(0, nk // 16, scat, 0)
            pltpu.sync_copy(st_v, st_hbm)
            pltpu.sync_copy(gs_v, gs_hbm)

    return k(p_flat, tok_flat, g_flat, dflt)


def _sc_gather_rows(table, idx, chunk):
    rows = idx.shape[0]
    d = table.shape[1]
    nw = 32
    per_w = rows // nw
    nch = per_w // chunk
    mesh = plsc.VectorSubcoreMesh(core_axis_name="c", subcore_axis_name="s")

    @functools.partial(
        pl.kernel,
        out_type=jax.ShapeDtypeStruct((rows, d), table.dtype),
        mesh=mesh,
        scratch_types=[
            pltpu.VMEM((chunk,), jnp.int32),
            pltpu.VMEM((chunk, d), table.dtype),
            pltpu.SemaphoreType.DMA,
        ],
    )
    def k(tab_hbm, idx_hbm, out_hbm, idx_v, rows_v, sem):
        wid = lax.axis_index("s") * 2 + lax.axis_index("c")

        def body(ch, _):
            base = wid * per_w + ch * chunk
            pltpu.sync_copy(idx_hbm.at[pl.ds(base, chunk)], idx_v)
            pltpu.async_copy(tab_hbm.at[idx_v], rows_v, sem).wait()
            pltpu.sync_copy(rows_v, out_hbm.at[pl.ds(base, chunk)])
            return 0

        lax.fori_loop(0, nch, body, 0)

    return k(table, idx)


def _shared_body(x_ref, wg_ref, wu_ref, wd_ref, out_ref, acc_ref,
                 wgb_ref, wub_ref, wdb_ref, *, nf, tb):
    f = pl.program_id(0)
    t = pl.program_id(1)

    @pl.when(t == 0)
    def _():
        wgb_ref[...] = wg_ref[...].astype(jnp.bfloat16)
        wub_ref[...] = wu_ref[...].astype(jnp.bfloat16)
        wdb_ref[...] = wd_ref[...].astype(jnp.bfloat16)

    x = x_ref[...].astype(jnp.bfloat16)
    g = lax.dot_general(
        x, wgb_ref[...], (((1,), (1,)), ((), ())),
        preferred_element_type=jnp.float32,
    ).astype(jnp.bfloat16)
    u = lax.dot_general(
        x, wub_ref[...], (((1,), (1,)), ((), ())),
        preferred_element_type=jnp.float32,
    ).astype(jnp.bfloat16)
    h = g * jax.nn.sigmoid(g) * u
    part = lax.dot_general(
        h, wdb_ref[...], (((1,), (1,)), ((), ())),
        preferred_element_type=jnp.float32,
    )
    rows = pl.ds(t * tb, tb)

    @pl.when(f == 0)
    def _():
        acc_ref[rows, :] = part

    @pl.when(f != 0)
    def _():
        acc_ref[rows, :] += part

    @pl.when(f == nf - 1)
    def _():
        out_ref[...] = acc_ref[rows, :]


def _grouped_body(be_ref, xs_ref, gs_ref, wg_ref, wu_ref, wd_ref, y_ref,
                  acc_ref, wgb_ref, wub_ref, wdb_ref, *, nf, tb, bh, ne):
    h_i = pl.program_id(0)
    f = pl.program_id(1)
    b = pl.program_id(2)
    gb = h_i * bh + b
    active = be_ref[gb] < ne
    fresh = (b == 0) | (be_ref[gb] != be_ref[jnp.maximum(gb - 1, 0)])

    @pl.when(active)
    def _():
        @pl.when(fresh)
        def _():
            wgb_ref[...] = wg_ref[0].astype(jnp.bfloat16)
            wub_ref[...] = wu_ref[0].astype(jnp.bfloat16)
            wdb_ref[...] = wd_ref[0].astype(jnp.bfloat16)

        x = xs_ref[...].astype(jnp.bfloat16)
        g = lax.dot_general(
            x, wgb_ref[...], (((1,), (1,)), ((), ())),
            preferred_element_type=jnp.float32,
        ).astype(jnp.bfloat16)
        u = lax.dot_general(
            x, wub_ref[...], (((1,), (1,)), ((), ())),
            preferred_element_type=jnp.float32,
        ).astype(jnp.bfloat16)
        hh = g * jax.nn.sigmoid(g) * u
        part = lax.dot_general(
            hh, wdb_ref[...], (((1,), (1,)), ((), ())),
            preferred_element_type=jnp.float32,
        )
        rows = pl.ds(b * tb, tb)

        @pl.when(f == 0)
        def _():
            acc_ref[rows, :] = part

        @pl.when(f != 0)
        def _():
            acc_ref[rows, :] += part

        @pl.when(f == nf - 1)
        def _():
            y_ref[...] = gs_ref[...] * acc_ref[rows, :]


def _sc_gather_flat(table, idx, chunk):
    rows = idx.shape[0]
    d = table.shape[1]
    nw = 32
    per_w = rows // nw
    nch = per_w // chunk
    mesh = plsc.VectorSubcoreMesh(core_axis_name="c", subcore_axis_name="s")

    @functools.partial(
        pl.kernel,
        out_type=jax.ShapeDtypeStruct((rows, d), table.dtype),
        mesh=mesh,
        scratch_types=[
            pltpu.VMEM((per_w,), jnp.int32),
            pltpu.VMEM((chunk, d), table.dtype),
            pltpu.VMEM((chunk, d), table.dtype),
            pltpu.SemaphoreType.DMA,
            pltpu.SemaphoreType.DMA,
            pltpu.SemaphoreType.DMA,
            pltpu.SemaphoreType.DMA,
        ],
    )
    def k(tab_hbm, idx_hbm, out_hbm, i_v, b0, b1, g0, g1, w0, w1):
        wid = lax.axis_index("s") * 2 + lax.axis_index("c")
        base = wid * per_w
        pltpu.sync_copy(idx_hbm.at[pl.ds(base, per_w)], i_v)
        bufs = (b0, b1)
        gsem = (g0, g1)
        wsem = (w0, w1)
        writes = [None, None]
        for ch in range(nch):
            s = ch % 2
            if writes[s] is not None:
                writes[s].wait()
            pltpu.async_copy(
                tab_hbm.at[i_v.at[pl.ds(ch * chunk, chunk)]], bufs[s], gsem[s]
            ).wait()
            writes[s] = pltpu.async_copy(
                bufs[s], out_hbm.at[pl.ds(base + ch * chunk, chunk)], wsem[s]
            )
        for wr in writes:
            if wr is not None:
                wr.wait()

    return k(table, idx)


def _combine_body(sh_ref, y0_ref, y1_ref, out_ref):
    out_ref[...] = sh_ref[...] + y0_ref[...] + y1_ref[...]


def kernel(X, W_router, expert_bias, Wg_s, Wu_s, Wd_s, Wg_e, Wu_e, Wd_e):
    batch, seq, d = X.shape
    n = batch * seq
    ne, dff, _ = Wg_e.shape
    xf = X.reshape(n, d)

    tb = min(256, n)
    nt = n // tb
    fb = min(1024, dff)
    nf = dff // fb
    tbg = min(512, n)
    nblk = (n * 2) // tbg + ne
    padn = nblk * tbg
    npart = 3 if nblk % 3 == 0 else (2 if nblk % 2 == 0 else 1)
    bh = nblk // npart

    topw, tope, r, cnt3 = pl.pallas_call(
        functools.partial(_router_body, n_exp=ne, tb=tb),
        grid=(nt,),
        in_specs=[
            pl.BlockSpec((tb, d), lambda t: (t, 0)),
            pl.BlockSpec((ne, d), lambda t: (0, 0)),
            pl.BlockSpec((1, ne), lambda t: (0, 0)),
        ],
        out_specs=[
            pl.BlockSpec((tb, 2), lambda t: (t, 0)),
            pl.BlockSpec((tb, 2), lambda t: (t, 0)),
            pl.BlockSpec((tb, 2), lambda t: (t, 0)),
            pl.BlockSpec((1, 1, ne), lambda t: (t, 0, 0)),
        ],
        out_shape=[
            jax.ShapeDtypeStruct((n, 2), jnp.float32),
            jax.ShapeDtypeStruct((n, 2), jnp.int32),
            jax.ShapeDtypeStruct((n, 2), jnp.int32),
            jax.ShapeDtypeStruct((nt, 1, ne), jnp.float32),
        ],
    )(xf, W_router, expert_bias.reshape(1, ne))

    p, be3 = pl.pallas_call(
        functools.partial(_dispatch_body, n_exp=ne, tb=tbg, nt=nt, nblk=nblk),
        grid=(nt,),
        in_specs=[
            pl.BlockSpec((nt, 1, ne), lambda t: (0, 0, 0)),
            pl.BlockSpec((tb, 2), lambda t: (t, 0)),
            pl.BlockSpec((tb, 2), lambda t: (t, 0)),
        ],
        out_specs=[
            pl.BlockSpec((tb, 2), lambda t: (t, 0)),
            pl.BlockSpec((1, 1, nblk), lambda t: (0, 0, 0)),
        ],
        out_shape=[
            jax.ShapeDtypeStruct((n, 2), jnp.int32),
            jax.ShapeDtypeStruct((1, 1, nblk), jnp.int32),
        ],
    )(cnt3, tope, r)

    tok_flat = jnp.arange(n * 2, dtype=jnp.int32) // 2
    dflt = jnp.arange(padn, dtype=jnp.int32) % n
    sorted_token, gates_sorted = _sc_permute(
        p.reshape(n * 2), tok_flat, topw.reshape(n * 2), dflt, padn
    )

    x_sorted = _sc_gather_flat(xf, sorted_token, chunk=padn // 32 // 8)

    shared = pl.pallas_call(
        functools.partial(_shared_body, nf=nf, tb=tb),
        grid=(nf, nt),
        in_specs=[
            pl.BlockSpec((tb, d), lambda f, t: (t, 0)),
            pl.BlockSpec((fb, d), lambda f, t: (f, 0)),
            pl.BlockSpec((fb, d), lambda f, t: (f, 0)),
            pl.BlockSpec((d, fb), lambda f, t: (0, f)),
        ],
        out_specs=pl.BlockSpec(
            (tb, d), lambda f, t: (jnp.where(f == nf - 1, t, 0), 0)
        ),
        out_shape=jax.ShapeDtypeStruct((n, d), jnp.float32),
        scratch_shapes=[
            pltpu.VMEM((n, d), jnp.float32),
            pltpu.VMEM((fb, d), jnp.bfloat16),
            pltpu.VMEM((fb, d), jnp.bfloat16),
            pltpu.VMEM((d, fb), jnp.bfloat16),
        ],
    )(xf, Wg_s, Wu_s, Wd_s)

    y = pl.pallas_call(
        functools.partial(_grouped_body, nf=nf, tb=tbg, bh=bh, ne=ne),
        grid_spec=pltpu.PrefetchScalarGridSpec(
            num_scalar_prefetch=1,
            grid=(npart, nf, bh),
            in_specs=[
                pl.BlockSpec((tbg, d), lambda h, f, b, be: (h * bh + b, 0)),
                pl.BlockSpec((tbg, 1), lambda h, f, b, be: (h * bh + b, 0)),
                pl.BlockSpec((1, fb, d),
                             lambda h, f, b, be: (jnp.minimum(be[h * bh + b], 7), f, 0)),
                pl.BlockSpec((1, fb, d),
                             lambda h, f, b, be: (jnp.minimum(be[h * bh + b], 7), f, 0)),
                pl.BlockSpec((1, d, fb),
                             lambda h, f, b, be: (jnp.minimum(be[h * bh + b], 7), 0, f)),
            ],
            out_specs=pl.BlockSpec(
                (tbg, d),
                lambda h, f, b, be: (jnp.where(f == nf - 1, h * bh + b, h * bh), 0),
            ),
            scratch_shapes=[
                pltpu.VMEM((bh * tbg, d), jnp.float32),
                pltpu.VMEM((fb, d), jnp.bfloat16),
                pltpu.VMEM((fb, d), jnp.bfloat16),
                pltpu.VMEM((d, fb), jnp.bfloat16),
            ],
        ),
        out_shape=jax.ShapeDtypeStruct((padn, d), jnp.float32),
    )(be3.reshape(nblk), x_sorted, gates_sorted.reshape(padn, 1),
      Wg_e, Wu_e, Wd_e)

    y0 = _sc_gather_flat(y, p[:, 0], chunk=32)
    y1 = _sc_gather_flat(y, p[:, 1], chunk=32)

    out = pl.pallas_call(
        _combine_body,
        grid=(nt,),
        in_specs=[
            pl.BlockSpec((tb, d), lambda t: (t, 0)),
            pl.BlockSpec((tb, d), lambda t: (t, 0)),
            pl.BlockSpec((tb, d), lambda t: (t, 0)),
        ],
        out_specs=pl.BlockSpec((tb, d), lambda t: (t, 0)),
        out_shape=jax.ShapeDtypeStruct((n, d), jnp.float32),
    )(shared, y0, y1)

    return out.reshape(batch, seq, d)


# speedup vs baseline: 1.1378x; 1.0243x over previous
import functools

import jax
import jax.numpy as jnp
from jax import lax
from jax.experimental import pallas as pl
from jax.experimental.pallas import tpu as pltpu
from jax.experimental.pallas import tpu_sc as plsc


def _router_body(x_ref, wr_ref, b_ref, topw_ref, tope_ref, r_ref, cnt_ref,
                 *, n_exp, tb):
    x = x_ref[...]
    wr = wr_ref[...]
    logits = lax.dot_general(
        x, wr, (((1,), (1,)), ((), ())), preferred_element_type=jnp.float32
    ) + b_ref[...]
    w = jax.nn.softmax(logits, axis=-1)
    lane = lax.broadcasted_iota(jnp.int32, w.shape, 1)
    rank = jnp.zeros_like(w)
    for j in range(n_exp):
        wj = w[:, j : j + 1]
        rank += (wj > w).astype(jnp.float32)
        rank += ((wj == w) & (j < lane)).astype(jnp.float32)
    on0 = rank < 0.5
    on1 = (rank >= 0.5) & (rank < 1.5)
    ew = jnp.exp(w)
    denom = jnp.sum(jnp.where(on0 | on1, ew, 0.0), axis=1, keepdims=True)
    w0 = jnp.sum(jnp.where(on0, ew, 0.0), axis=1, keepdims=True) / denom
    w1 = jnp.sum(jnp.where(on1, ew, 0.0), axis=1, keepdims=True) / denom
    e0 = jnp.sum(jnp.where(on0, lane, 0), axis=1, keepdims=True)
    e1 = jnp.sum(jnp.where(on1, lane, 0), axis=1, keepdims=True)
    topw_ref[...] = jnp.concatenate([w0, w1], axis=1)
    tope_ref[...] = jnp.concatenate([e0, e1], axis=1)
    occ = (on0 | on1).astype(jnp.bfloat16)
    row_i = lax.broadcasted_iota(jnp.int32, (tb, tb), 0)
    col_i = lax.broadcasted_iota(jnp.int32, (tb, tb), 1)
    ltr = (row_i > col_i).astype(jnp.bfloat16)
    cum = lax.dot_general(
        ltr, occ, (((1,), (0,)), ((), ())), preferred_element_type=jnp.float32
    )
    r0 = jnp.sum(jnp.where(on0, cum, 0.0), axis=1, keepdims=True)
    r1 = jnp.sum(jnp.where(on1, cum, 0.0), axis=1, keepdims=True)
    r_ref[...] = jnp.concatenate([r0, r1], axis=1).astype(jnp.int32)
    cnt_ref[...] = jnp.sum(occ.astype(jnp.float32), axis=0, keepdims=True)[None]


def _dispatch_body(cnt_ref, tope_ref, r_ref, p_ref, be_ref,
                   *, n_exp, tb, nt, nblk):
    t = pl.program_id(0)
    cnt = cnt_ref[:, 0, :]
    c = jnp.sum(cnt, axis=0, keepdims=True)
    row = lax.broadcasted_iota(jnp.int32, cnt.shape, 0)
    base = jnp.sum(jnp.where(row < t, cnt, 0.0), axis=0, keepdims=True)
    pb = jnp.floor((c + (tb - 1)) / tb)
    lane = lax.broadcasted_iota(jnp.int32, (1, n_exp), 1)
    start = jnp.zeros((1, n_exp), jnp.float32)
    for e in range(n_exp - 1):
        start += jnp.where(lane > e, pb[0:1, e : e + 1] * tb, 0.0)
    sb = start + base
    tope = tope_ref[...]
    r = r_ref[...].astype(jnp.float32)
    lane_tb = lax.broadcasted_iota(jnp.int32, (tope.shape[0], n_exp), 1)
    ps = []
    for k in range(2):
        onek = tope[:, k : k + 1] == lane_tb
        pk = jnp.sum(jnp.where(onek, sb, 0.0), axis=1, keepdims=True)
        ps.append(pk + r[:, k : k + 1])
    p_ref[...] = jnp.concatenate(ps, axis=1).astype(jnp.int32)

    @pl.when(t == 0)
    def _():
        lane_b = lax.broadcasted_iota(jnp.int32, (1, nblk), 1)
        acc = jnp.zeros((1, nblk), jnp.int32)
        run = pb[0:1, 0:1]
        for e in range(1, n_exp + 1):
            acc += (lane_b >= run.astype(jnp.int32)).astype(jnp.int32)
            if e < n_exp:
                run = run + pb[0:1, e : e + 1]
        be_ref[...] = acc[None]


def _sc_permute(p_flat, tok_flat, g_flat, dflt, padn):
    nk = p_flat.shape[0]
    mesh = plsc.VectorSubcoreMesh(core_axis_name="c", subcore_axis_name="s")

    @functools.partial(
        pl.kernel,
        out_type=[
            jax.ShapeDtypeStruct((padn,), jnp.int32),
            jax.ShapeDtypeStruct((padn,), jnp.float32),
        ],
        mesh=mesh,
        scratch_types=[
            pltpu.VMEM((nk,), jnp.int32),
            pltpu.VMEM((nk,), jnp.int32),
            pltpu.VMEM((nk,), jnp.float32),
            pltpu.VMEM((padn,), jnp.int32),
            pltpu.VMEM((padn,), jnp.float32),
        ],
        compiler_params=pltpu.CompilerParams(needs_layout_passes=False),
    )
    def k(p_hbm, tok_hbm, g_hbm, dflt_hbm, st_hbm, gs_hbm, p_v, t_v, g_v, st_v, gs_v):
        cid = lax.axis_index("c")
        sid = lax.axis_index("s")

        @pl.when((cid == 0) & (sid == 0))
        def _():
            pltpu.sync_copy(p_hbm, p_v)
            pltpu.sync_copy(tok_hbm, t_v)
            pltpu.sync_copy(g_hbm, g_v)
            pltpu.sync_copy(dflt_hbm, st_v)
            zf = jnp.zeros((16,), jnp.float32)

            def memset(i, _):
                gs_v[pl.ds(i * 16, 16)] = zf
                return 0

            lax.fori_loop(0, padn // 16, memset, 0)

            def scat(j, _):
                idx = p_v[pl.ds(j * 16, 16)]
                plsc.store_scatter(st_v, [idx], t_v[pl.ds(j * 16, 16)])
                plsc.store_scatter(gs_v, [idx], g_v[pl.ds(j * 16, 16)])
                return 0

            lax.fori_loop(0, nk // 16, scat, 0)
            pltpu.sync_copy(st_v, st_hbm)
            pltpu.sync_copy(gs_v, gs_hbm)

    return k(p_flat, tok_flat, g_flat, dflt)


def _sc_gather_rows(table, idx, chunk):
    rows = idx.shape[0]
    d = table.shape[1]
    nw = 32
    per_w = rows // nw
    nch = per_w // chunk
    mesh = plsc.VectorSubcoreMesh(core_axis_name="c", subcore_axis_name="s")

    @functools.partial(
        pl.kernel,
        out_type=jax.ShapeDtypeStruct((rows, d), table.dtype),
        mesh=mesh,
        scratch_types=[
            pltpu.VMEM((chunk,), jnp.int32),
            pltpu.VMEM((chunk, d), table.dtype),
            pltpu.SemaphoreType.DMA,
        ],
    )
    def k(tab_hbm, idx_hbm, out_hbm, idx_v, rows_v, sem):
        wid = lax.axis_index("s") * 2 + lax.axis_index("c")

        def body(ch, _):
            base = wid * per_w + ch * chunk
            pltpu.sync_copy(idx_hbm.at[pl.ds(base, chunk)], idx_v)
            pltpu.async_copy(tab_hbm.at[idx_v], rows_v, sem).wait()
            pltpu.sync_copy(rows_v, out_hbm.at[pl.ds(base, chunk)])
            return 0

        lax.fori_loop(0, nch, body, 0)

    return k(table, idx)


def _shared_body(x_ref, wg_ref, wu_ref, wd_ref, out_ref, acc_ref,
                 wgb_ref, wub_ref, wdb_ref, *, nf, tb):
    f = pl.program_id(0)
    t = pl.program_id(1)

    @pl.when(t == 0)
    def _():
        wgb_ref[...] = wg_ref[...].astype(jnp.bfloat16)
        wub_ref[...] = wu_ref[...].astype(jnp.bfloat16)
        wdb_ref[...] = wd_ref[...].astype(jnp.bfloat16)

    x = x_ref[...].astype(jnp.bfloat16)
    g = lax.dot_general(
        x, wgb_ref[...], (((1,), (1,)), ((), ())),
        preferred_element_type=jnp.float32,
    ).astype(jnp.bfloat16)
    u = lax.dot_general(
        x, wub_ref[...], (((1,), (1,)), ((), ())),
        preferred_element_type=jnp.float32,
    ).astype(jnp.bfloat16)
    h = g * jax.nn.sigmoid(g) * u
    part = lax.dot_general(
        h, wdb_ref[...], (((1,), (1,)), ((), ())),
        preferred_element_type=jnp.float32,
    )
    rows = pl.ds(t * tb, tb)

    @pl.when(f == 0)
    def _():
        acc_ref[rows, :] = part

    @pl.when(f != 0)
    def _():
        acc_ref[rows, :] += part

    @pl.when(f == nf - 1)
    def _():
        out_ref[...] = acc_ref[rows, :]


def _grouped_body(be_ref, xs_ref, gs_ref, wg_ref, wu_ref, wd_ref, y_ref,
                  acc_ref, wgb_ref, wub_ref, wdb_ref, *, nf, tb, bh, ne):
    h_i = pl.program_id(0)
    f = pl.program_id(1)
    b = pl.program_id(2)
    gb = h_i * bh + b
    active = be_ref[gb] < ne
    fresh = (b == 0) | (be_ref[gb] != be_ref[jnp.maximum(gb - 1, 0)])

    @pl.when(active)
    def _():
        @pl.when(fresh)
        def _():
            wgb_ref[...] = wg_ref[0].astype(jnp.bfloat16)
            wub_ref[...] = wu_ref[0].astype(jnp.bfloat16)
            wdb_ref[...] = wd_ref[0].astype(jnp.bfloat16)

        x = xs_ref[...].astype(jnp.bfloat16)
        g = lax.dot_general(
            x, wgb_ref[...], (((1,), (1,)), ((), ())),
            preferred_element_type=jnp.float32,
        ).astype(jnp.bfloat16)
        u = lax.dot_general(
            x, wub_ref[...], (((1,), (1,)), ((), ())),
            preferred_element_type=jnp.float32,
        ).astype(jnp.bfloat16)
        hh = g * jax.nn.sigmoid(g) * u
        part = lax.dot_general(
            hh, wdb_ref[...], (((1,), (1,)), ((), ())),
            preferred_element_type=jnp.float32,
        )
        rows = pl.ds(b * tb, tb)

        @pl.when(f == 0)
        def _():
            acc_ref[rows, :] = part

        @pl.when(f != 0)
        def _():
            acc_ref[rows, :] += part

        @pl.when(f == nf - 1)
        def _():
            y_ref[...] = gs_ref[...] * acc_ref[rows, :]


def _sc_gather_flat(table, idx, chunk):
    rows = idx.shape[0]
    d = table.shape[1]
    nw = 32
    per_w = rows // nw
    nch = per_w // chunk
    mesh = plsc.VectorSubcoreMesh(core_axis_name="c", subcore_axis_name="s")

    @functools.partial(
        pl.kernel,
        out_type=jax.ShapeDtypeStruct((rows, d), table.dtype),
        mesh=mesh,
        scratch_types=[
            pltpu.VMEM((per_w,), jnp.int32),
            pltpu.VMEM((chunk, d), table.dtype),
            pltpu.VMEM((chunk, d), table.dtype),
            pltpu.SemaphoreType.DMA,
            pltpu.SemaphoreType.DMA,
            pltpu.SemaphoreType.DMA,
            pltpu.SemaphoreType.DMA,
        ],
    )
    def k(tab_hbm, idx_hbm, out_hbm, i_v, b0, b1, g0, g1, w0, w1):
        wid = lax.axis_index("s") * 2 + lax.axis_index("c")
        base = wid * per_w
        pltpu.sync_copy(idx_hbm.at[pl.ds(base, per_w)], i_v)
        bufs = (b0, b1)
        gsem = (g0, g1)
        wsem = (w0, w1)
        writes = [None, None]
        for ch in range(nch):
            s = ch % 2
            if writes[s] is not None:
                writes[s].wait()
            pltpu.async_copy(
                tab_hbm.at[i_v.at[pl.ds(ch * chunk, chunk)]], bufs[s], gsem[s]
            ).wait()
            writes[s] = pltpu.async_copy(
                bufs[s], out_hbm.at[pl.ds(base + ch * chunk, chunk)], wsem[s]
            )
        for wr in writes:
            if wr is not None:
                wr.wait()

    return k(table, idx)


def _combine_body(sh_ref, y0_ref, y1_ref, out_ref):
    out_ref[...] = sh_ref[...] + y0_ref[...] + y1_ref[...]


def kernel(X, W_router, expert_bias, Wg_s, Wu_s, Wd_s, Wg_e, Wu_e, Wd_e):
    batch, seq, d = X.shape
    n = batch * seq
    ne, dff, _ = Wg_e.shape
    xf = X.reshape(n, d)

    tb = min(256, n)
    nt = n // tb
    fb = min(1024, dff)
    nf = dff // fb
    tbg = min(512, n)
    nblk = (n * 2) // tbg + ne
    padn = nblk * tbg
    npart = 3 if nblk % 3 == 0 else (2 if nblk % 2 == 0 else 1)
    bh = nblk // npart

    topw, tope, r, cnt3 = pl.pallas_call(
        functools.partial(_router_body, n_exp=ne, tb=tb),
        grid=(nt,),
        in_specs=[
            pl.BlockSpec((tb, d), lambda t: (t, 0)),
            pl.BlockSpec((ne, d), lambda t: (0, 0)),
            pl.BlockSpec((1, ne), lambda t: (0, 0)),
        ],
        out_specs=[
            pl.BlockSpec((tb, 2), lambda t: (t, 0)),
            pl.BlockSpec((tb, 2), lambda t: (t, 0)),
            pl.BlockSpec((tb, 2), lambda t: (t, 0)),
            pl.BlockSpec((1, 1, ne), lambda t: (t, 0, 0)),
        ],
        out_shape=[
            jax.ShapeDtypeStruct((n, 2), jnp.float32),
            jax.ShapeDtypeStruct((n, 2), jnp.int32),
            jax.ShapeDtypeStruct((n, 2), jnp.int32),
            jax.ShapeDtypeStruct((nt, 1, ne), jnp.float32),
        ],
    )(xf, W_router, expert_bias.reshape(1, ne))

    p, be3 = pl.pallas_call(
        functools.partial(_dispatch_body, n_exp=ne, tb=tbg, nt=nt, nblk=nblk),
        grid=(nt,),
        in_specs=[
            pl.BlockSpec((nt, 1, ne), lambda t: (0, 0, 0)),
            pl.BlockSpec((tb, 2), lambda t: (t, 0)),
            pl.BlockSpec((tb, 2), lambda t: (t, 0)),
        ],
        out_specs=[
            pl.BlockSpec((tb, 2), lambda t: (t, 0)),
            pl.BlockSpec((1, 1, nblk), lambda t: (0, 0, 0)),
        ],
        out_shape=[
            jax.ShapeDtypeStruct((n, 2), jnp.int32),
            jax.ShapeDtypeStruct((1, 1, nblk), jnp.int32),
        ],
    )(cnt3, tope, r)

    tok_flat = jnp.arange(n * 2, dtype=jnp.int32) // 2
    dflt = jnp.arange(padn, dtype=jnp.int32) % n
    sorted_token, gates_sorted = _sc_permute(
        p.reshape(n * 2), tok_flat, topw.reshape(n * 2), dflt, padn
    )

    x_sorted = _sc_gather_flat(xf, sorted_token, chunk=padn // 32 // 8)

    tbs = min(512, n)
    nts = n // tbs
    shared = pl.pallas_call(
        functools.partial(_shared_body, nf=nf, tb=tbs),
        grid=(nf, nts),
        in_specs=[
            pl.BlockSpec((tbs, d), lambda f, t: (t, 0)),
            pl.BlockSpec((fb, d), lambda f, t: (f, 0)),
            pl.BlockSpec((fb, d), lambda f, t: (f, 0)),
            pl.BlockSpec((d, fb), lambda f, t: (0, f)),
        ],
        out_specs=pl.BlockSpec(
            (tbs, d), lambda f, t: (jnp.where(f == nf - 1, t, 0), 0)
        ),
        out_shape=jax.ShapeDtypeStruct((n, d), jnp.float32),
        scratch_shapes=[
            pltpu.VMEM((n, d), jnp.float32),
            pltpu.VMEM((fb, d), jnp.bfloat16),
            pltpu.VMEM((fb, d), jnp.bfloat16),
            pltpu.VMEM((d, fb), jnp.bfloat16),
        ],
    )(xf, Wg_s, Wu_s, Wd_s)

    y = pl.pallas_call(
        functools.partial(_grouped_body, nf=nf, tb=tbg, bh=bh, ne=ne),
        grid_spec=pltpu.PrefetchScalarGridSpec(
            num_scalar_prefetch=1,
            grid=(npart, nf, bh),
            in_specs=[
                pl.BlockSpec((tbg, d), lambda h, f, b, be: (h * bh + b, 0)),
                pl.BlockSpec((tbg, 1), lambda h, f, b, be: (h * bh + b, 0)),
                pl.BlockSpec((1, fb, d),
                             lambda h, f, b, be: (jnp.minimum(be[h * bh + b], 7), f, 0)),
                pl.BlockSpec((1, fb, d),
                             lambda h, f, b, be: (jnp.minimum(be[h * bh + b], 7), f, 0)),
                pl.BlockSpec((1, d, fb),
                             lambda h, f, b, be: (jnp.minimum(be[h * bh + b], 7), 0, f)),
            ],
            out_specs=pl.BlockSpec(
                (tbg, d),
                lambda h, f, b, be: (jnp.where(f == nf - 1, h * bh + b, h * bh), 0),
            ),
            scratch_shapes=[
                pltpu.VMEM((bh * tbg, d), jnp.float32),
                pltpu.VMEM((fb, d), jnp.bfloat16),
                pltpu.VMEM((fb, d), jnp.bfloat16),
                pltpu.VMEM((d, fb), jnp.bfloat16),
            ],
        ),
        out_shape=jax.ShapeDtypeStruct((padn, d), jnp.float32),
    )(be3.reshape(nblk), x_sorted, gates_sorted.reshape(padn, 1),
      Wg_e, Wu_e, Wd_e)

    pcat = jnp.concatenate([p[:, 0], p[:, 1]])
    ycat = _sc_gather_flat(y, pcat, chunk=32)

    out = pl.pallas_call(
        _combine_body,
        grid=(nt,),
        in_specs=[
            pl.BlockSpec((tb, d), lambda t: (t, 0)),
            pl.BlockSpec((tb, d), lambda t: (t, 0)),
            pl.BlockSpec((tb, d), lambda t: (t + n // tb, 0)),
        ],
        out_specs=pl.BlockSpec((tb, d), lambda t: (t, 0)),
        out_shape=jax.ShapeDtypeStruct((n, d), jnp.float32),
    )(shared, ycat, ycat)

    return out.reshape(batch, seq, d)


# speedup vs baseline: 1.1431x; 1.0047x over previous
import functools

import jax
import jax.numpy as jnp
from jax import lax
from jax.experimental import pallas as pl
from jax.experimental.pallas import tpu as pltpu
from jax.experimental.pallas import tpu_sc as plsc


def _routerdisp_body(x_ref, wr_ref, b_ref, topw_ref, p_ref, be_ref,
                     cnt_s, e_s, r_s, *, n_exp, tb, nt, nblk, tbg):
    ph = pl.program_id(0)
    t = pl.program_id(1)
    rows = pl.ds(t * tb, tb)

    @pl.when(ph == 0)
    def _():
        x = x_ref[...]
        wr = wr_ref[...]
        logits = lax.dot_general(
            x, wr, (((1,), (1,)), ((), ())), preferred_element_type=jnp.float32
        ) + b_ref[...]
        w = jax.nn.softmax(logits, axis=-1)
        lane = lax.broadcasted_iota(jnp.int32, w.shape, 1)
        rank = jnp.zeros_like(w)
        for j in range(n_exp):
            wj = w[:, j : j + 1]
            rank += (wj > w).astype(jnp.float32)
            rank += ((wj == w) & (j < lane)).astype(jnp.float32)
        on0 = rank < 0.5
        on1 = (rank >= 0.5) & (rank < 1.5)
        ew = jnp.exp(w)
        denom = jnp.sum(jnp.where(on0 | on1, ew, 0.0), axis=1, keepdims=True)
        w0 = jnp.sum(jnp.where(on0, ew, 0.0), axis=1, keepdims=True) / denom
        w1 = jnp.sum(jnp.where(on1, ew, 0.0), axis=1, keepdims=True) / denom
        e0 = jnp.sum(jnp.where(on0, lane, 0), axis=1, keepdims=True)
        e1 = jnp.sum(jnp.where(on1, lane, 0), axis=1, keepdims=True)
        topw_ref[...] = jnp.concatenate([w0, w1], axis=1)
        e_s[rows, :] = jnp.concatenate([e0, e1], axis=1)
        occ = (on0 | on1).astype(jnp.bfloat16)
        row_i = lax.broadcasted_iota(jnp.int32, (tb, tb), 0)
        col_i = lax.broadcasted_iota(jnp.int32, (tb, tb), 1)
        ltr = (row_i > col_i).astype(jnp.bfloat16)
        cum = lax.dot_general(
            ltr, occ, (((1,), (0,)), ((), ())),
            preferred_element_type=jnp.float32,
        )
        r0 = jnp.sum(jnp.where(on0, cum, 0.0), axis=1, keepdims=True)
        r1 = jnp.sum(jnp.where(on1, cum, 0.0), axis=1, keepdims=True)
        r_s[rows, :] = jnp.concatenate([r0, r1], axis=1)
        cnt_s[pl.ds(t, 1), :] = jnp.sum(occ.astype(jnp.float32), axis=0,
                                        keepdims=True)

    @pl.when(ph == 1)
    def _():
        cnt = cnt_s[...]
        c = jnp.sum(cnt, axis=0, keepdims=True)
        row = lax.broadcasted_iota(jnp.int32, cnt.shape, 0)
        base = jnp.sum(jnp.where(row < t, cnt, 0.0), axis=0, keepdims=True)
        pb = jnp.floor((c + (tbg - 1)) / tbg)
        lane = lax.broadcasted_iota(jnp.int32, (1, n_exp), 1)
        start = jnp.zeros((1, n_exp), jnp.float32)
        for e in range(n_exp - 1):
            start += jnp.where(lane > e, pb[0:1, e : e + 1] * tbg, 0.0)
        sb = start + base
        tope = e_s[rows, :]
        r = r_s[rows, :]
        lane_tb = lax.broadcasted_iota(jnp.int32, (tb, n_exp), 1)
        ps = []
        for k in range(2):
            onek = tope[:, k : k + 1] == lane_tb
            pk = jnp.sum(jnp.where(onek, sb, 0.0), axis=1, keepdims=True)
            ps.append(pk + r[:, k : k + 1])
        p_ref[...] = jnp.concatenate(ps, axis=1).astype(jnp.int32)

        @pl.when(t == 0)
        def _():
            lane_b = lax.broadcasted_iota(jnp.int32, (1, nblk), 1)
            acc = jnp.zeros((1, nblk), jnp.int32)
            run = pb[0:1, 0:1]
            for e in range(1, n_exp + 1):
                acc += (lane_b >= run.astype(jnp.int32)).astype(jnp.int32)
                if e < n_exp:
                    run = run + pb[0:1, e : e + 1]
            be_ref[...] = acc[None]


def _sc_permute(p_flat, tok_flat, g_flat, dflt, padn):
    nk = p_flat.shape[0]
    mesh = plsc.VectorSubcoreMesh(core_axis_name="c", subcore_axis_name="s")

    @functools.partial(
        pl.kernel,
        out_type=[
            jax.ShapeDtypeStruct((padn,), jnp.int32),
            jax.ShapeDtypeStruct((padn,), jnp.float32),
        ],
        mesh=mesh,
        scratch_types=[
            pltpu.VMEM((nk,), jnp.int32),
            pltpu.VMEM((nk,), jnp.int32),
            pltpu.VMEM((nk,), jnp.float32),
            pltpu.VMEM((padn,), jnp.int32),
            pltpu.VMEM((padn,), jnp.float32),
        ],
        compiler_params=pltpu.CompilerParams(needs_layout_passes=False),
    )
    def k(p_hbm, tok_hbm, g_hbm, dflt_hbm, st_hbm, gs_hbm, p_v, t_v, g_v, st_v, gs_v):
        cid = lax.axis_index("c")
        sid = lax.axis_index("s")

        @pl.when((cid == 0) & (sid == 0))
        def _():
            pltpu.sync_copy(p_hbm, p_v)
            pltpu.sync_copy(tok_hbm, t_v)
            pltpu.sync_copy(g_hbm, g_v)
            pltpu.sync_copy(dflt_hbm, st_v)
            zf = jnp.zeros((16,), jnp.float32)

            def memset(i, _):
                gs_v[pl.ds(i * 16, 16)] = zf
                return 0

            lax.fori_loop(0, padn // 16, memset, 0)

            def scat(j, _):
                idx = p_v[pl.ds(j * 16, 16)]
                plsc.store_scatter(st_v, [idx], t_v[pl.ds(j * 16, 16)])
                plsc.store_scatter(gs_v, [idx], g_v[pl.ds(j * 16, 16)])
                return 0

            lax.fori_loop(0, nk // 16, scat, 0)
            pltpu.sync_copy(st_v, st_hbm)
            pltpu.sync_copy(gs_v, gs_hbm)

    return k(p_flat, tok_flat, g_flat, dflt)


def _sc_gather_rows(table, idx, chunk):
    rows = idx.shape[0]
    d = table.shape[1]
    nw = 32
    per_w = rows // nw
    nch = per_w // chunk
    mesh = plsc.VectorSubcoreMesh(core_axis_name="c", subcore_axis_name="s")

    @functools.partial(
        pl.kernel,
        out_type=jax.ShapeDtypeStruct((rows, d), table.dtype),
        mesh=mesh,
        scratch_types=[
            pltpu.VMEM((chunk,), jnp.int32),
            pltpu.VMEM((chunk, d), table.dtype),
            pltpu.SemaphoreType.DMA,
        ],
    )
    def k(tab_hbm, idx_hbm, out_hbm, idx_v, rows_v, sem):
        wid = lax.axis_index("s") * 2 + lax.axis_index("c")

        def body(ch, _):
            base = wid * per_w + ch * chunk
            pltpu.sync_copy(idx_hbm.at[pl.ds(base, chunk)], idx_v)
            pltpu.async_copy(tab_hbm.at[idx_v], rows_v, sem).wait()
            pltpu.sync_copy(rows_v, out_hbm.at[pl.ds(base, chunk)])
            return 0

        lax.fori_loop(0, nch, body, 0)

    return k(table, idx)


def _shared_body(x_ref, wg_ref, wu_ref, wd_ref, out_ref, acc_ref,
                 wgb_ref, wub_ref, wdb_ref, *, nf, tb):
    f = pl.program_id(0)
    t = pl.program_id(1)

    @pl.when(t == 0)
    def _():
        wgb_ref[...] = wg_ref[...].astype(jnp.bfloat16)
        wub_ref[...] = wu_ref[...].astype(jnp.bfloat16)
        wdb_ref[...] = wd_ref[...].astype(jnp.bfloat16)

    x = x_ref[...].astype(jnp.bfloat16)
    g = lax.dot_general(
        x, wgb_ref[...], (((1,), (1,)), ((), ())),
        preferred_element_type=jnp.float32,
    ).astype(jnp.bfloat16)
    u = lax.dot_general(
        x, wub_ref[...], (((1,), (1,)), ((), ())),
        preferred_element_type=jnp.float32,
    ).astype(jnp.bfloat16)
    h = g * jax.nn.sigmoid(g) * u
    part = lax.dot_general(
        h, wdb_ref[...], (((1,), (1,)), ((), ())),
        preferred_element_type=jnp.float32,
    )
    rows = pl.ds(t * tb, tb)

    @pl.when(f == 0)
    def _():
        acc_ref[rows, :] = part

    @pl.when(f != 0)
    def _():
        acc_ref[rows, :] += part

    @pl.when(f == nf - 1)
    def _():
        out_ref[...] = acc_ref[rows, :]


def _grouped_body(be_ref, xs_ref, gs_ref, wg_ref, wu_ref, wd_ref, y_ref,
                  acc_ref, wgb_ref, wub_ref, wdb_ref, *, nf, tb, bh, ne):
    h_i = pl.program_id(0)
    f = pl.program_id(1)
    b = pl.program_id(2)
    gb = h_i * bh + b
    active = be_ref[gb] < ne
    fresh = (b == 0) | (be_ref[gb] != be_ref[jnp.maximum(gb - 1, 0)])

    @pl.when(active)
    def _():
        @pl.when(fresh)
        def _():
            wgb_ref[...] = wg_ref[0].astype(jnp.bfloat16)
            wub_ref[...] = wu_ref[0].astype(jnp.bfloat16)
            wdb_ref[...] = wd_ref[0].astype(jnp.bfloat16)

        x = xs_ref[...].astype(jnp.bfloat16)
        g = lax.dot_general(
            x, wgb_ref[...], (((1,), (1,)), ((), ())),
            preferred_element_type=jnp.float32,
        ).astype(jnp.bfloat16)
        u = lax.dot_general(
            x, wub_ref[...], (((1,), (1,)), ((), ())),
            preferred_element_type=jnp.float32,
        ).astype(jnp.bfloat16)
        hh = g * jax.nn.sigmoid(g) * u
        part = lax.dot_general(
            hh, wdb_ref[...], (((1,), (1,)), ((), ())),
            preferred_element_type=jnp.float32,
        )
        rows = pl.ds(b * tb, tb)

        @pl.when(f == 0)
        def _():
            acc_ref[rows, :] = part

        @pl.when(f != 0)
        def _():
            acc_ref[rows, :] += part

        @pl.when(f == nf - 1)
        def _():
            y_ref[...] = gs_ref[...] * acc_ref[rows, :]


def _sc_gather_flat(table, idx, chunk):
    rows = idx.shape[0]
    d = table.shape[1]
    nw = 32
    per_w = rows // nw
    nch = per_w // chunk
    mesh = plsc.VectorSubcoreMesh(core_axis_name="c", subcore_axis_name="s")

    @functools.partial(
        pl.kernel,
        out_type=jax.ShapeDtypeStruct((rows, d), table.dtype),
        mesh=mesh,
        scratch_types=[
            pltpu.VMEM((per_w,), jnp.int32),
            pltpu.VMEM((chunk, d), table.dtype),
            pltpu.VMEM((chunk, d), table.dtype),
            pltpu.SemaphoreType.DMA,
            pltpu.SemaphoreType.DMA,
            pltpu.SemaphoreType.DMA,
            pltpu.SemaphoreType.DMA,
        ],
    )
    def k(tab_hbm, idx_hbm, out_hbm, i_v, b0, b1, g0, g1, w0, w1):
        wid = lax.axis_index("s") * 2 + lax.axis_index("c")
        base = wid * per_w
        pltpu.sync_copy(idx_hbm.at[pl.ds(base, per_w)], i_v)
        bufs = (b0, b1)
        gsem = (g0, g1)
        wsem = (w0, w1)
        writes = [None, None]
        for ch in range(nch):
            s = ch % 2
            if writes[s] is not None:
                writes[s].wait()
            pltpu.async_copy(
                tab_hbm.at[i_v.at[pl.ds(ch * chunk, chunk)]], bufs[s], gsem[s]
            ).wait()
            writes[s] = pltpu.async_copy(
                bufs[s], out_hbm.at[pl.ds(base + ch * chunk, chunk)], wsem[s]
            )
        for wr in writes:
            if wr is not None:
                wr.wait()

    return k(table, idx)


def _combine_body(sh_ref, y0_ref, y1_ref, out_ref):
    out_ref[...] = sh_ref[...] + y0_ref[...] + y1_ref[...]


def kernel(X, W_router, expert_bias, Wg_s, Wu_s, Wd_s, Wg_e, Wu_e, Wd_e):
    batch, seq, d = X.shape
    n = batch * seq
    ne, dff, _ = Wg_e.shape
    xf = X.reshape(n, d)

    tb = min(256, n)
    nt = n // tb
    fb = min(1024, dff)
    nf = dff // fb
    tbg = min(512, n)
    nblk = (n * 2) // tbg + ne
    padn = nblk * tbg
    npart = 3 if nblk % 3 == 0 else (2 if nblk % 2 == 0 else 1)
    bh = nblk // npart

    topw, p, be3 = pl.pallas_call(
        functools.partial(_routerdisp_body, n_exp=ne, tb=tb, nt=nt,
                          nblk=nblk, tbg=tbg),
        grid=(2, nt),
        in_specs=[
            pl.BlockSpec((tb, d), lambda ph, t: (jnp.where(ph == 0, t, 0), 0)),
            pl.BlockSpec((ne, d), lambda ph, t: (0, 0)),
            pl.BlockSpec((1, ne), lambda ph, t: (0, 0)),
        ],
        out_specs=[
            pl.BlockSpec((tb, 2), lambda ph, t: (jnp.where(ph == 0, t, nt - 1), 0)),
            pl.BlockSpec((tb, 2), lambda ph, t: (jnp.where(ph == 1, t, nt - 1), 0)),
            pl.BlockSpec((1, 1, nblk), lambda ph, t: (0, 0, 0)),
        ],
        out_shape=[
            jax.ShapeDtypeStruct((n, 2), jnp.float32),
            jax.ShapeDtypeStruct((n, 2), jnp.int32),
            jax.ShapeDtypeStruct((1, 1, nblk), jnp.int32),
        ],
        scratch_shapes=[
            pltpu.VMEM((nt, ne), jnp.float32),
            pltpu.VMEM((n, 2), jnp.int32),
            pltpu.VMEM((n, 2), jnp.float32),
        ],
    )(xf, W_router, expert_bias.reshape(1, ne))

    tok_flat = jnp.arange(n * 2, dtype=jnp.int32) // 2
    dflt = jnp.arange(padn, dtype=jnp.int32) % n
    sorted_token, gates_sorted = _sc_permute(
        p.reshape(n * 2), tok_flat, topw.reshape(n * 2), dflt, padn
    )

    x_sorted = _sc_gather_flat(xf, sorted_token, chunk=padn // 32 // 8)

    tbs = min(512, n)
    nts = n // tbs
    shared = pl.pallas_call(
        functools.partial(_shared_body, nf=nf, tb=tbs),
        grid=(nf, nts),
        in_specs=[
            pl.BlockSpec((tbs, d), lambda f, t: (t, 0)),
            pl.BlockSpec((fb, d), lambda f, t: (f, 0)),
            pl.BlockSpec((fb, d), lambda f, t: (f, 0)),
            pl.BlockSpec((d, fb), lambda f, t: (0, f)),
        ],
        out_specs=pl.BlockSpec(
            (tbs, d), lambda f, t: (jnp.where(f == nf - 1, t, 0), 0)
        ),
        out_shape=jax.ShapeDtypeStruct((n, d), jnp.float32),
        scratch_shapes=[
            pltpu.VMEM((n, d), jnp.float32),
            pltpu.VMEM((fb, d), jnp.bfloat16),
            pltpu.VMEM((fb, d), jnp.bfloat16),
            pltpu.VMEM((d, fb), jnp.bfloat16),
        ],
    )(xf, Wg_s, Wu_s, Wd_s)

    y = pl.pallas_call(
        functools.partial(_grouped_body, nf=nf, tb=tbg, bh=bh, ne=ne),
        grid_spec=pltpu.PrefetchScalarGridSpec(
            num_scalar_prefetch=1,
            grid=(npart, nf, bh),
            in_specs=[
                pl.BlockSpec((tbg, d), lambda h, f, b, be: (h * bh + b, 0)),
                pl.BlockSpec((tbg, 1), lambda h, f, b, be: (h * bh + b, 0)),
                pl.BlockSpec((1, fb, d),
                             lambda h, f, b, be: (jnp.minimum(be[h * bh + b], 7), f, 0)),
                pl.BlockSpec((1, fb, d),
                             lambda h, f, b, be: (jnp.minimum(be[h * bh + b], 7), f, 0)),
                pl.BlockSpec((1, d, fb),
                             lambda h, f, b, be: (jnp.minimum(be[h * bh + b], 7), 0, f)),
            ],
            out_specs=pl.BlockSpec(
                (tbg, d),
                lambda h, f, b, be: (jnp.where(f == nf - 1, h * bh + b, h * bh), 0),
            ),
            scratch_shapes=[
                pltpu.VMEM((bh * tbg, d), jnp.float32),
                pltpu.VMEM((fb, d), jnp.bfloat16),
                pltpu.VMEM((fb, d), jnp.bfloat16),
                pltpu.VMEM((d, fb), jnp.bfloat16),
            ],
        ),
        out_shape=jax.ShapeDtypeStruct((padn, d), jnp.float32),
    )(be3.reshape(nblk), x_sorted, gates_sorted.reshape(padn, 1),
      Wg_e, Wu_e, Wd_e)

    pcat = jnp.concatenate([p[:, 0], p[:, 1]])
    ycat = _sc_gather_flat(y, pcat, chunk=32)

    out = pl.pallas_call(
        _combine_body,
        grid=(nt,),
        in_specs=[
            pl.BlockSpec((tb, d), lambda t: (t, 0)),
            pl.BlockSpec((tb, d), lambda t: (t, 0)),
            pl.BlockSpec((tb, d), lambda t: (t + n // tb, 0)),
        ],
        out_specs=pl.BlockSpec((tb, d), lambda t: (t, 0)),
        out_shape=jax.ShapeDtypeStruct((n, d), jnp.float32),
    )(shared, ycat, ycat)

    return out.reshape(batch, seq, d)
